# baseline (speedup 1.0000x reference)
import math

import jax
import jax.numpy as jnp
from jax import lax
from jax.experimental import pallas as pl
from jax.experimental.pallas import tpu as pltpu
from jax.experimental.pallas import tpu_sc as plsc

_BN_SCALE = 1.0 / math.sqrt(1.0 + 1e-5)
_K = 128
_NBUF = 2


def _sc_geometry(E):
  info = plsc.get_sparse_core_info()
  NC, NS = info.num_cores, info.num_subcores
  NW = NC * NS
  iters = -(-E // (_K * NW))
  return NC, NS, NW, iters, iters * _K * NW


def _make_sc_agg(N, D, E_pad, mode):
  NC, NS, NW, iters, e_chk = _sc_geometry(E_pad)
  assert e_chk == E_pad
  NA = N + 8
  rows_per_tile = (N // NS) & ~7
  rows_extra = N - rows_per_tile * NS
  assert rows_extra % 8 == 0 and rows_extra + 8 <= _K
  gather = mode == "agg"
  CH = iters + _NBUF

  mesh = plsc.VectorSubcoreMesh(core_axis_name="c", subcore_axis_name="s")
  out_type = [jax.ShapeDtypeStruct((NC, N, D), jnp.float32)]
  if gather:
    scratch = [
        pltpu.VMEM((CH, _K), jnp.int32),
        pltpu.VMEM((_K, D), jnp.float32),
        pltpu.VMEM((_K, D), jnp.float32),
        pltpu.VMEM((_K,), jnp.int32),
        pltpu.VMEM_SHARED((NA, D), jnp.float32),
        pltpu.SemaphoreType.DMA, pltpu.SemaphoreType.DMA,
    ]
  else:
    scratch = [
        pltpu.VMEM((CH, _K), jnp.int32),
        pltpu.VMEM((_K, D), jnp.float32),
        pltpu.VMEM_SHARED((NA, D), jnp.float32),
    ]

  def body(*refs):
    if gather:
      (h_hbm, src_hbm, dst_hbm, out_hbm, src_v, r0, r1, db, acc,
       gs0, gs1) = refs
      rows, gsems = [r0, r1], [gs0, gs1]
      rows0 = r0
    else:
      dst_hbm, out_hbm, dst_v, rows0, acc = refs
    cid = lax.axis_index("c")
    sid = lax.axis_index("s")
    wid = sid * NC + cid

    if gather:
      pltpu.sync_copy(src_hbm.at[wid], src_v)
    else:
      pltpu.sync_copy(dst_hbm.at[wid], dst_v)

    def fill_rows(val):
      vec = jnp.full((16,), val, jnp.float32)

      def w(i, _):
        rows0[i // (D // 16), pl.ds((i % (D // 16)) * 16, 16)] = vec
        return 0
      lax.fori_loop(0, _K * (D // 16), w, 0)

    fill_rows(0.0)

    row0 = sid * rows_per_tile
    n_full = rows_per_tile // _K
    tail = rows_per_tile - n_full * _K
    for b in range(n_full):
      pltpu.sync_copy(rows0, acc.at[pl.ds(row0 + b * _K, _K)])
    if tail:
      pltpu.sync_copy(rows0.at[pl.ds(0, tail)],
                      acc.at[pl.ds(row0 + n_full * _K, tail)])

    @pl.when(sid == 0)
    def _():
      pltpu.sync_copy(rows0.at[pl.ds(0, rows_extra + 8)],
                      acc.at[pl.ds(rows_per_tile * NS, rows_extra + 8)])

    if not gather:
      fill_rows(1.0)
    plsc.subcore_barrier()

    if gather:
      for b in range(_NBUF):
        pltpu.async_copy(h_hbm.at[src_v.at[b]], rows[b], gsems[b])

      def step(c, b):
        pltpu.sync_copy(dst_hbm.at[pl.ds((wid * CH + c) * _K, _K)], db)
        pltpu.make_async_copy(h_hbm.at[src_v.at[c]], rows[b],
                              gsems[b]).wait()
        pltpu.sync_copy(rows[b], acc.at[db], add=True)
        pltpu.async_copy(h_hbm.at[src_v.at[c + _NBUF]], rows[b], gsems[b])

      n_outer = iters // _NBUF

      def outer(g, _):
        for b in range(_NBUF):
          step(g * _NBUF + b, b)
        return 0
      lax.fori_loop(0, n_outer, outer, 0)
      for c in range(n_outer * _NBUF, iters):
        step(c, c % _NBUF)
      for c in range(iters, iters + _NBUF):
        b = c % _NBUF
        pltpu.make_async_copy(h_hbm.at[src_v.at[c]], rows[b],
                              gsems[b]).wait()
    else:
      def chunk(c, _):
        pltpu.sync_copy(rows0, acc.at[dst_v.at[c]], add=True)
        return 0
      lax.fori_loop(0, iters, chunk, 0)
    plsc.subcore_barrier()

    pltpu.sync_copy(acc.at[pl.ds(row0, rows_per_tile)],
                    out_hbm.at[cid, pl.ds(row0, rows_per_tile)])
    if rows_extra:
      @pl.when(sid == 0)
      def _():
        pltpu.sync_copy(acc.at[pl.ds(rows_per_tile * NS, rows_extra)],
                        out_hbm.at[cid, pl.ds(rows_per_tile * NS, rows_extra)])

  return pl.kernel(body, out_type=out_type, mesh=mesh, scratch_types=scratch)


def _dense_layer(p, cnt, h, Wl, bl, Wr, g, be, relu):
  N, D = h.shape
  TILE = 1000

  def body(p0_ref, p1_ref, c0_ref, c1_ref, h_ref, wl_ref, bl_ref, wr_ref,
           g_ref, be_ref, o_ref):
    cnt_t = c0_ref[:, :1] + c1_ref[:, :1]
    inv = 1.0 / jnp.maximum(cnt_t, 1.0)
    mean = (p0_ref[...] + p1_ref[...]) * inv
    acc = lax.dot_general(mean, wl_ref[...], (((1,), (1,)), ((), ())),
                          preferred_element_type=jnp.float32)
    acc = acc + lax.dot_general(h_ref[...], wr_ref[...],
                                (((1,), (1,)), ((), ())),
                                preferred_element_type=jnp.float32)
    acc = acc + bl_ref[...]
    if relu:
      acc = jnp.maximum(acc, 0.0)
    o_ref[...] = acc * (g_ref[...] * _BN_SCALE) + be_ref[...]

  grid = (N // TILE,)
  row_spec = pl.BlockSpec((TILE, D), lambda i: (i, 0))
  cnt_spec = pl.BlockSpec((TILE, D), lambda i: (i, 0))
  full = lambda shape: pl.BlockSpec(shape, lambda i: (0,) * len(shape))
  return pl.pallas_call(
      body,
      grid=grid,
      in_specs=[row_spec, row_spec, cnt_spec, cnt_spec, row_spec,
                full((D, D)), full((1, D)), full((D, D)),
                full((1, D)), full((1, D))],
      out_specs=row_spec,
      out_shape=jax.ShapeDtypeStruct((N, D), jnp.float32),
  )(p[0], p[1], cnt[0], cnt[1], h, Wl, bl.reshape(1, D), Wr,
    g.reshape(1, D), be.reshape(1, D))


def _dense_final(p, cnt, h, Wl, bl, Wr, g, be, Wf, bf):
  N, D = h.shape
  TILE = 1000

  def body(p0_ref, p1_ref, c0_ref, c1_ref, h_ref, wl_ref, bl_ref, wr_ref,
           g_ref, be_ref, wf_ref, bf_ref, o_ref):
    cnt_t = c0_ref[:, :1] + c1_ref[:, :1]
    inv = 1.0 / jnp.maximum(cnt_t, 1.0)
    mean = (p0_ref[...] + p1_ref[...]) * inv
    acc = lax.dot_general(mean, wl_ref[...], (((1,), (1,)), ((), ())),
                          preferred_element_type=jnp.float32)
    acc = acc + lax.dot_general(h_ref[...], wr_ref[...],
                                (((1,), (1,)), ((), ())),
                                preferred_element_type=jnp.float32)
    acc = acc + bl_ref[...]
    acc = acc * (g_ref[...] * _BN_SCALE) + be_ref[...]
    logit = jnp.sum(acc * wf_ref[...], axis=1, keepdims=True) + bf_ref[0, 0]
    o_ref[...] = 1.0 / (1.0 + jnp.exp(-logit))

  grid = (N // TILE,)
  row_spec = pl.BlockSpec((TILE, D), lambda i: (i, 0))
  cnt_spec = pl.BlockSpec((TILE, D), lambda i: (i, 0))
  full = lambda shape: pl.BlockSpec(shape, lambda i: (0,) * len(shape))
  return pl.pallas_call(
      body,
      grid=grid,
      in_specs=[row_spec, row_spec, cnt_spec, cnt_spec, row_spec,
                full((D, D)), full((1, D)), full((D, D)),
                full((1, D)), full((1, D)), full((1, D)),
                pl.BlockSpec(memory_space=pltpu.SMEM)],
      out_specs=pl.BlockSpec((TILE, 1), lambda i: (i, 0)),
      out_shape=jax.ShapeDtypeStruct((N, 1), jnp.float32),
  )(p[0], p[1], cnt[0], cnt[1], h, Wl, bl.reshape(1, D), Wr,
    g.reshape(1, D), be.reshape(1, D), Wf.reshape(1, D), bf.reshape(1, 1))


def _pad_edges(src, dst, N, NW, iters):
  pad = NW * iters * _K - src.shape[0]
  src_p = jnp.concatenate([src, jnp.zeros((pad,), jnp.int32)])
  dst_p = jnp.concatenate([dst, jnp.full((pad,), N, jnp.int32)])
  src_p = src_p.reshape(NW, iters, _K)
  dst_p = dst_p.reshape(NW, iters, _K)
  src_p = jnp.concatenate(
      [src_p, jnp.zeros((NW, _NBUF, _K), jnp.int32)], axis=1)
  dst_p = jnp.concatenate(
      [dst_p, jnp.full((NW, _NBUF, _K), N, jnp.int32)], axis=1)
  return src_p, dst_p


@jax.jit
def kernel(x, adj_t, Wl0, bl0, Wr0, Wl1, bl1, Wr1, Wl2, bl2, Wr2,
           g0, be0, g1, be1, g2, be2, Wf, bf):
  N, D = x.shape
  E = adj_t.shape[1]
  _, _, NW, iters, E_pad = _sc_geometry(E)
  src, dst = _pad_edges(adj_t[0], adj_t[1], N, NW, iters)

  count = _make_sc_agg(N, D, E_pad, mode="count")
  agg = _make_sc_agg(N, D, E_pad, mode="agg")

  dst_flat = dst.reshape(-1)
  (cnt,) = count(dst)
  (p,) = agg(x, src, dst_flat)
  h1 = _dense_layer(p, cnt, x, Wl0, bl0, Wr0, g0, be0, relu=True)
  (p,) = agg(h1, src, dst_flat)
  h2 = _dense_layer(p, cnt, h1, Wl1, bl1, Wr1, g1, be1, relu=True)
  (p,) = agg(h2, src, dst_flat)
  return _dense_final(p, cnt, h2, Wl2, bl2, Wr2, g2, be2, Wf, bf)


# speedup vs baseline: 1.7294x; 1.7294x over previous
import math

import jax
import jax.numpy as jnp
from jax import lax
from jax.experimental import pallas as pl
from jax.experimental.pallas import tpu as pltpu
from jax.experimental.pallas import tpu_sc as plsc

_BN_SCALE = 1.0 / math.sqrt(1.0 + 1e-5)
_K = 128
_NBUF = 2


def _sc_geometry(E):
  info = plsc.get_sparse_core_info()
  NC, NS = info.num_cores, info.num_subcores
  NW = NC * NS
  iters = -(-E // (_K * NW))
  return NC, NS, NW, iters, iters * _K * NW


def _make_sc_agg(N, D, E_pad, mode):
  NC, NS, NW, iters, e_chk = _sc_geometry(E_pad)
  assert e_chk == E_pad
  NA = N + 8
  rows_per_tile = (N // NS) & ~7
  rows_extra = N - rows_per_tile * NS
  assert rows_extra % 8 == 0 and rows_extra + 8 <= _K
  gather = mode == "agg"
  CH = iters + _NBUF

  mesh = plsc.VectorSubcoreMesh(core_axis_name="c", subcore_axis_name="s")
  out_type = [jax.ShapeDtypeStruct((NC, N, D), jnp.float32)]
  if gather:
    scratch = [
        pltpu.VMEM((CH, _K), jnp.int32),
        pltpu.VMEM((_K, D), jnp.float32),
        pltpu.VMEM((_K, D), jnp.float32),
        pltpu.VMEM((_K,), jnp.int32),
        pltpu.VMEM((_K,), jnp.int32),
        pltpu.VMEM_SHARED((NA, D), jnp.float32),
        pltpu.SemaphoreType.DMA, pltpu.SemaphoreType.DMA,
        pltpu.SemaphoreType.DMA, pltpu.SemaphoreType.DMA,
    ]
  else:
    scratch = [
        pltpu.VMEM((CH, _K), jnp.int32),
        pltpu.VMEM((_K, D), jnp.float32),
        pltpu.VMEM_SHARED((NA, D), jnp.float32),
    ]

  def body(*refs):
    if gather:
      (h_hbm, src_hbm, dst_hbm, out_hbm, src_v, r0, r1, db0, db1, acc,
       gs0, gs1, ss0, ss1) = refs
      rows, dstb = [r0, r1], [db0, db1]
      gsems, ssems = [gs0, gs1], [ss0, ss1]
      rows0 = r0
    else:
      dst_hbm, out_hbm, dst_v, rows0, acc = refs
    cid = lax.axis_index("c")
    sid = lax.axis_index("s")
    wid = sid * NC + cid

    if gather:
      pltpu.sync_copy(src_hbm.at[wid], src_v)
    else:
      pltpu.sync_copy(dst_hbm.at[wid], dst_v)

    def fill_rows(val):
      vec = jnp.full((16,), val, jnp.float32)

      def w(i, _):
        rows0[i // (D // 16), pl.ds((i % (D // 16)) * 16, 16)] = vec
        return 0
      lax.fori_loop(0, _K * (D // 16), w, 0)

    fill_rows(0.0)

    row0 = sid * rows_per_tile
    n_full = rows_per_tile // _K
    tail = rows_per_tile - n_full * _K
    for b in range(n_full):
      pltpu.sync_copy(rows0, acc.at[pl.ds(row0 + b * _K, _K)])
    if tail:
      pltpu.sync_copy(rows0.at[pl.ds(0, tail)],
                      acc.at[pl.ds(row0 + n_full * _K, tail)])

    @pl.when(sid == 0)
    def _():
      pltpu.sync_copy(rows0.at[pl.ds(0, rows_extra + 8)],
                      acc.at[pl.ds(rows_per_tile * NS, rows_extra + 8)])

    if not gather:
      fill_rows(1.0)
    plsc.subcore_barrier()

    if gather:
      def g_copy(c, b):
        return pltpu.make_async_copy(h_hbm.at[src_v.at[c]], rows[b],
                                     gsems[b])


      pltpu.sync_copy(dst_hbm.at[pl.ds((wid * CH) * _K, _K)], dstb[0])
      pltpu.sync_copy(dst_hbm.at[pl.ds((wid * CH + 1) * _K, _K)], dstb[1])
      pltpu.async_copy(h_hbm.at[src_v.at[0]], rows[0], gsems[0])
      pltpu.async_copy(h_hbm.at[src_v.at[1]], rows[1], gsems[1])

      def step(c, b, first, last):
        g_copy(c, b).wait()
        pltpu.async_copy(rows[b], acc.at[dstb[b]], ssems[b], add=True)
        if not first:
          pltpu.make_async_copy(rows[1 - b], acc.at[dstb[1 - b]],
                                ssems[1 - b]).wait()
          if not last:
            pltpu.sync_copy(
                dst_hbm.at[pl.ds((wid * CH + c + 1) * _K, _K)], dstb[1 - b])
            pltpu.async_copy(h_hbm.at[src_v.at[c + 1]], rows[1 - b],
                             gsems[1 - b])

      step(0, 0, True, False)
      n_outer = (iters - 2) // _NBUF

      def outer(g, _):
        for b in range(_NBUF):
          step(1 + g * _NBUF + b, (1 + b) % _NBUF, False, False)
        return 0
      lax.fori_loop(0, n_outer, outer, 0)
      for c in range(1 + n_outer * _NBUF, iters):
        step(c, c % _NBUF, False, c == iters - 1)
      pltpu.make_async_copy(rows[(iters - 1) % _NBUF],
                            acc.at[dstb[(iters - 1) % _NBUF]],
                            ssems[(iters - 1) % _NBUF]).wait()
    else:
      def chunk(c, _):
        pltpu.sync_copy(rows0, acc.at[dst_v.at[c]], add=True)
        return 0
      lax.fori_loop(0, iters, chunk, 0)
    plsc.subcore_barrier()

    pltpu.sync_copy(acc.at[pl.ds(row0, rows_per_tile)],
                    out_hbm.at[cid, pl.ds(row0, rows_per_tile)])
    if rows_extra:
      @pl.when(sid == 0)
      def _():
        pltpu.sync_copy(acc.at[pl.ds(rows_per_tile * NS, rows_extra)],
                        out_hbm.at[cid, pl.ds(rows_per_tile * NS, rows_extra)])

  return pl.kernel(body, out_type=out_type, mesh=mesh, scratch_types=scratch)


def _dense_layer(p, cnt, h, Wl, bl, Wr, g, be, relu):
  N, D = h.shape
  TILE = 1000

  def body(p0_ref, p1_ref, c0_ref, c1_ref, h_ref, wl_ref, bl_ref, wr_ref,
           g_ref, be_ref, o_ref):
    cnt_t = c0_ref[:, :1] + c1_ref[:, :1]
    inv = 1.0 / jnp.maximum(cnt_t, 1.0)
    mean = (p0_ref[...] + p1_ref[...]) * inv
    acc = lax.dot_general(mean, wl_ref[...], (((1,), (1,)), ((), ())),
                          preferred_element_type=jnp.float32)
    acc = acc + lax.dot_general(h_ref[...], wr_ref[...],
                                (((1,), (1,)), ((), ())),
                                preferred_element_type=jnp.float32)
    acc = acc + bl_ref[...]
    if relu:
      acc = jnp.maximum(acc, 0.0)
    o_ref[...] = acc * (g_ref[...] * _BN_SCALE) + be_ref[...]

  grid = (N // TILE,)
  row_spec = pl.BlockSpec((TILE, D), lambda i: (i, 0))
  cnt_spec = pl.BlockSpec((TILE, D), lambda i: (i, 0))
  full = lambda shape: pl.BlockSpec(shape, lambda i: (0,) * len(shape))
  return pl.pallas_call(
      body,
      grid=grid,
      in_specs=[row_spec, row_spec, cnt_spec, cnt_spec, row_spec,
                full((D, D)), full((1, D)), full((D, D)),
                full((1, D)), full((1, D))],
      out_specs=row_spec,
      out_shape=jax.ShapeDtypeStruct((N, D), jnp.float32),
  )(p[0], p[1], cnt[0], cnt[1], h, Wl, bl.reshape(1, D), Wr,
    g.reshape(1, D), be.reshape(1, D))


def _dense_final(p, cnt, h, Wl, bl, Wr, g, be, Wf, bf):
  N, D = h.shape
  TILE = 1000

  def body(p0_ref, p1_ref, c0_ref, c1_ref, h_ref, wl_ref, bl_ref, wr_ref,
           g_ref, be_ref, wf_ref, bf_ref, o_ref):
    cnt_t = c0_ref[:, :1] + c1_ref[:, :1]
    inv = 1.0 / jnp.maximum(cnt_t, 1.0)
    mean = (p0_ref[...] + p1_ref[...]) * inv
    acc = lax.dot_general(mean, wl_ref[...], (((1,), (1,)), ((), ())),
                          preferred_element_type=jnp.float32)
    acc = acc + lax.dot_general(h_ref[...], wr_ref[...],
                                (((1,), (1,)), ((), ())),
                                preferred_element_type=jnp.float32)
    acc = acc + bl_ref[...]
    acc = acc * (g_ref[...] * _BN_SCALE) + be_ref[...]
    logit = jnp.sum(acc * wf_ref[...], axis=1, keepdims=True) + bf_ref[0, 0]
    o_ref[...] = 1.0 / (1.0 + jnp.exp(-logit))

  grid = (N // TILE,)
  row_spec = pl.BlockSpec((TILE, D), lambda i: (i, 0))
  cnt_spec = pl.BlockSpec((TILE, D), lambda i: (i, 0))
  full = lambda shape: pl.BlockSpec(shape, lambda i: (0,) * len(shape))
  return pl.pallas_call(
      body,
      grid=grid,
      in_specs=[row_spec, row_spec, cnt_spec, cnt_spec, row_spec,
                full((D, D)), full((1, D)), full((D, D)),
                full((1, D)), full((1, D)), full((1, D)),
                pl.BlockSpec(memory_space=pltpu.SMEM)],
      out_specs=pl.BlockSpec((TILE, 1), lambda i: (i, 0)),
      out_shape=jax.ShapeDtypeStruct((N, 1), jnp.float32),
  )(p[0], p[1], cnt[0], cnt[1], h, Wl, bl.reshape(1, D), Wr,
    g.reshape(1, D), be.reshape(1, D), Wf.reshape(1, D), bf.reshape(1, 1))


def _pad_edges(src, dst, N, NW, iters):
  pad = NW * iters * _K - src.shape[0]
  src_p = jnp.concatenate([src, jnp.zeros((pad,), jnp.int32)])
  dst_p = jnp.concatenate([dst, jnp.full((pad,), N, jnp.int32)])
  src_p = src_p.reshape(NW, iters, _K)
  dst_p = dst_p.reshape(NW, iters, _K)
  src_p = jnp.concatenate(
      [src_p, jnp.zeros((NW, _NBUF, _K), jnp.int32)], axis=1)
  dst_p = jnp.concatenate(
      [dst_p, jnp.full((NW, _NBUF, _K), N, jnp.int32)], axis=1)
  return src_p, dst_p


@jax.jit
def kernel(x, adj_t, Wl0, bl0, Wr0, Wl1, bl1, Wr1, Wl2, bl2, Wr2,
           g0, be0, g1, be1, g2, be2, Wf, bf):
  N, D = x.shape
  E = adj_t.shape[1]
  _, _, NW, iters, E_pad = _sc_geometry(E)
  src, dst = _pad_edges(adj_t[0], adj_t[1], N, NW, iters)

  count = _make_sc_agg(N, D, E_pad, mode="count")
  agg = _make_sc_agg(N, D, E_pad, mode="agg")

  dst_flat = dst.reshape(-1)
  (cnt,) = count(dst)
  (p,) = agg(x, src, dst_flat)
  h1 = _dense_layer(p, cnt, x, Wl0, bl0, Wr0, g0, be0, relu=True)
  (p,) = agg(h1, src, dst_flat)
  h2 = _dense_layer(p, cnt, h1, Wl1, bl1, Wr1, g1, be1, relu=True)
  (p,) = agg(h2, src, dst_flat)
  return _dense_final(p, cnt, h2, Wl2, bl2, Wr2, g2, be2, Wf, bf)


# speedup vs baseline: 1.7890x; 1.0345x over previous
import math

import jax
import jax.numpy as jnp
from jax import lax
from jax.experimental import pallas as pl
from jax.experimental.pallas import tpu as pltpu
from jax.experimental.pallas import tpu_sc as plsc

_BN_SCALE = 1.0 / math.sqrt(1.0 + 1e-5)
_K = 128
_NBUF = 2


def _sc_geometry(E):
  info = plsc.get_sparse_core_info()
  NC, NS = info.num_cores, info.num_subcores
  NW = NC * NS
  iters = -(-E // (_K * NW))
  return NC, NS, NW, iters, iters * _K * NW


def _make_sc_agg(N, D, E_pad, mode):
  NC, NS, NW, iters, e_chk = _sc_geometry(E_pad)
  assert e_chk == E_pad
  NA = N + 8
  rows_per_tile = (N // NS) & ~7
  rows_extra = N - rows_per_tile * NS
  assert rows_extra % 8 == 0 and rows_extra + 8 <= _K
  gather = mode == "agg"
  CH = iters + _NBUF

  mesh = plsc.VectorSubcoreMesh(core_axis_name="c", subcore_axis_name="s")
  out_type = [jax.ShapeDtypeStruct((NC, N, D), jnp.float32)]
  if gather:
    scratch = [
        pltpu.VMEM((CH, _K), jnp.int32),
        pltpu.VMEM((_K, D), jnp.float32),
        pltpu.VMEM((_K, D), jnp.float32),
        pltpu.VMEM((_K,), jnp.int32),
        pltpu.VMEM((_K,), jnp.int32),
        pltpu.VMEM_SHARED((NA, D), jnp.float32),
        pltpu.SemaphoreType.DMA, pltpu.SemaphoreType.DMA,
        pltpu.SemaphoreType.DMA, pltpu.SemaphoreType.DMA,
    ]
  else:
    scratch = [
        pltpu.VMEM((CH, _K), jnp.int32),
        pltpu.VMEM((_K, D), jnp.float32),
        pltpu.VMEM_SHARED((NA, D), jnp.float32),
        pltpu.SemaphoreType.DMA, pltpu.SemaphoreType.DMA,
    ]

  def body(*refs):
    if gather:
      (h_hbm, src_hbm, dst_hbm, out_hbm, src_v, r0, r1, db0, db1, acc,
       gs0, gs1, ss0, ss1) = refs
      rows, dstb = [r0, r1], [db0, db1]
      gsems, ssems = [gs0, gs1], [ss0, ss1]
      rows0 = r0
    else:
      dst_hbm, out_hbm, dst_v, rows0, acc, cs0, cs1 = refs
      csems = [cs0, cs1]
    cid = lax.axis_index("c")
    sid = lax.axis_index("s")
    wid = sid * NC + cid

    if gather:
      pltpu.sync_copy(src_hbm.at[wid], src_v)
    else:
      pltpu.sync_copy(dst_hbm.at[wid], dst_v)

    def fill_rows(val):
      vec = jnp.full((16,), val, jnp.float32)

      def w(i, _):
        rows0[i // (D // 16), pl.ds((i % (D // 16)) * 16, 16)] = vec
        return 0
      lax.fori_loop(0, _K * (D // 16), w, 0)

    fill_rows(0.0)

    row0 = sid * rows_per_tile
    n_full = rows_per_tile // _K
    tail = rows_per_tile - n_full * _K
    for b in range(n_full):
      pltpu.sync_copy(rows0, acc.at[pl.ds(row0 + b * _K, _K)])
    if tail:
      pltpu.sync_copy(rows0.at[pl.ds(0, tail)],
                      acc.at[pl.ds(row0 + n_full * _K, tail)])

    @pl.when(sid == 0)
    def _():
      pltpu.sync_copy(rows0.at[pl.ds(0, rows_extra + 8)],
                      acc.at[pl.ds(rows_per_tile * NS, rows_extra + 8)])

    if not gather:
      fill_rows(1.0)
    plsc.subcore_barrier()

    if gather:
      def g_copy(c, b):
        return pltpu.make_async_copy(h_hbm.at[src_v.at[c]], rows[b],
                                     gsems[b])


      pltpu.sync_copy(dst_hbm.at[pl.ds((wid * CH) * _K, _K)], dstb[0])
      pltpu.sync_copy(dst_hbm.at[pl.ds((wid * CH + 1) * _K, _K)], dstb[1])
      pltpu.async_copy(h_hbm.at[src_v.at[0]], rows[0], gsems[0])
      pltpu.async_copy(h_hbm.at[src_v.at[1]], rows[1], gsems[1])

      def step(c, b, first, last):
        g_copy(c, b).wait()
        pltpu.async_copy(rows[b], acc.at[dstb[b]], ssems[b], add=True)
        if not first:
          pltpu.make_async_copy(rows[1 - b], acc.at[dstb[1 - b]],
                                ssems[1 - b]).wait()
          if not last:
            pltpu.sync_copy(
                dst_hbm.at[pl.ds((wid * CH + c + 1) * _K, _K)], dstb[1 - b])
            pltpu.async_copy(h_hbm.at[src_v.at[c + 1]], rows[1 - b],
                             gsems[1 - b])

      step(0, 0, True, False)
      n_outer = (iters - 2) // _NBUF

      def outer(g, _):
        for b in range(_NBUF):
          step(1 + g * _NBUF + b, (1 + b) % _NBUF, False, False)
        return 0
      lax.fori_loop(0, n_outer, outer, 0)
      for c in range(1 + n_outer * _NBUF, iters):
        step(c, c % _NBUF, False, c == iters - 1)
      pltpu.make_async_copy(rows[(iters - 1) % _NBUF],
                            acc.at[dstb[(iters - 1) % _NBUF]],
                            ssems[(iters - 1) % _NBUF]).wait()
    else:
      def cstep(c, b):
        pltpu.async_copy(rows0, acc.at[dst_v.at[c]], csems[b], add=True)
        pltpu.make_async_copy(rows0, acc.at[dst_v.at[c - 1]],
                              csems[1 - b]).wait()

      pltpu.async_copy(rows0, acc.at[dst_v.at[0]], csems[0], add=True)
      n_outer = (iters - 1) // 2

      def pair(g, _):
        cstep(1 + g * 2, 1)
        cstep(2 + g * 2, 0)
        return 0
      lax.fori_loop(0, n_outer, pair, 0)
      for c in range(1 + n_outer * 2, iters):
        cstep(c, c % 2)
      pltpu.make_async_copy(rows0, acc.at[dst_v.at[iters - 1]],
                            csems[(iters - 1) % 2]).wait()
    plsc.subcore_barrier()

    pltpu.sync_copy(acc.at[pl.ds(row0, rows_per_tile)],
                    out_hbm.at[cid, pl.ds(row0, rows_per_tile)])
    if rows_extra:
      @pl.when(sid == 0)
      def _():
        pltpu.sync_copy(acc.at[pl.ds(rows_per_tile * NS, rows_extra)],
                        out_hbm.at[cid, pl.ds(rows_per_tile * NS, rows_extra)])

  return pl.kernel(body, out_type=out_type, mesh=mesh, scratch_types=scratch)


def _dense_layer(p, cnt, h, Wl, bl, Wr, g, be, relu):
  N, D = h.shape
  TILE = 1000

  def body(p0_ref, p1_ref, c0_ref, c1_ref, h_ref, wl_ref, bl_ref, wr_ref,
           g_ref, be_ref, o_ref):
    cnt_t = c0_ref[:, :1] + c1_ref[:, :1]
    inv = 1.0 / jnp.maximum(cnt_t, 1.0)
    mean = (p0_ref[...] + p1_ref[...]) * inv
    acc = lax.dot_general(mean, wl_ref[...], (((1,), (1,)), ((), ())),
                          preferred_element_type=jnp.float32)
    acc = acc + lax.dot_general(h_ref[...], wr_ref[...],
                                (((1,), (1,)), ((), ())),
                                preferred_element_type=jnp.float32)
    acc = acc + bl_ref[...]
    if relu:
      acc = jnp.maximum(acc, 0.0)
    o_ref[...] = acc * (g_ref[...] * _BN_SCALE) + be_ref[...]

  grid = (N // TILE,)
  row_spec = pl.BlockSpec((TILE, D), lambda i: (i, 0))
  cnt_spec = pl.BlockSpec((TILE, D), lambda i: (i, 0))
  full = lambda shape: pl.BlockSpec(shape, lambda i: (0,) * len(shape))
  return pl.pallas_call(
      body,
      grid=grid,
      in_specs=[row_spec, row_spec, cnt_spec, cnt_spec, row_spec,
                full((D, D)), full((1, D)), full((D, D)),
                full((1, D)), full((1, D))],
      out_specs=row_spec,
      out_shape=jax.ShapeDtypeStruct((N, D), jnp.float32),
  )(p[0], p[1], cnt[0], cnt[1], h, Wl, bl.reshape(1, D), Wr,
    g.reshape(1, D), be.reshape(1, D))


def _dense_final(p, cnt, h, Wl, bl, Wr, g, be, Wf, bf):
  N, D = h.shape
  TILE = 1000

  def body(p0_ref, p1_ref, c0_ref, c1_ref, h_ref, wl_ref, bl_ref, wr_ref,
           g_ref, be_ref, wf_ref, bf_ref, o_ref):
    cnt_t = c0_ref[:, :1] + c1_ref[:, :1]
    inv = 1.0 / jnp.maximum(cnt_t, 1.0)
    mean = (p0_ref[...] + p1_ref[...]) * inv
    acc = lax.dot_general(mean, wl_ref[...], (((1,), (1,)), ((), ())),
                          preferred_element_type=jnp.float32)
    acc = acc + lax.dot_general(h_ref[...], wr_ref[...],
                                (((1,), (1,)), ((), ())),
                                preferred_element_type=jnp.float32)
    acc = acc + bl_ref[...]
    acc = acc * (g_ref[...] * _BN_SCALE) + be_ref[...]
    logit = jnp.sum(acc * wf_ref[...], axis=1, keepdims=True) + bf_ref[0, 0]
    o_ref[...] = 1.0 / (1.0 + jnp.exp(-logit))

  grid = (N // TILE,)
  row_spec = pl.BlockSpec((TILE, D), lambda i: (i, 0))
  cnt_spec = pl.BlockSpec((TILE, D), lambda i: (i, 0))
  full = lambda shape: pl.BlockSpec(shape, lambda i: (0,) * len(shape))
  return pl.pallas_call(
      body,
      grid=grid,
      in_specs=[row_spec, row_spec, cnt_spec, cnt_spec, row_spec,
                full((D, D)), full((1, D)), full((D, D)),
                full((1, D)), full((1, D)), full((1, D)),
                pl.BlockSpec(memory_space=pltpu.SMEM)],
      out_specs=pl.BlockSpec((TILE, 1), lambda i: (i, 0)),
      out_shape=jax.ShapeDtypeStruct((N, 1), jnp.float32),
  )(p[0], p[1], cnt[0], cnt[1], h, Wl, bl.reshape(1, D), Wr,
    g.reshape(1, D), be.reshape(1, D), Wf.reshape(1, D), bf.reshape(1, 1))


def _pad_edges(src, dst, N, NW, iters):
  pad = NW * iters * _K - src.shape[0]
  src_p = jnp.concatenate([src, jnp.zeros((pad,), jnp.int32)])
  dst_p = jnp.concatenate(
      [dst, N + (jnp.arange(pad, dtype=jnp.int32) % 8)])
  src_p = src_p.reshape(iters, NW, _K).transpose(1, 0, 2)
  dst_p = dst_p.reshape(iters, NW, _K).transpose(1, 0, 2)
  src_p = jnp.concatenate(
      [src_p, jnp.zeros((NW, _NBUF, _K), jnp.int32)], axis=1)
  dst_p = jnp.concatenate(
      [dst_p, jnp.full((NW, _NBUF, _K), N, jnp.int32)], axis=1)
  return src_p, dst_p


@jax.jit
def kernel(x, adj_t, Wl0, bl0, Wr0, Wl1, bl1, Wr1, Wl2, bl2, Wr2,
           g0, be0, g1, be1, g2, be2, Wf, bf):
  N, D = x.shape
  E = adj_t.shape[1]
  _, _, NW, iters, E_pad = _sc_geometry(E)
  src, dst = _pad_edges(adj_t[0], adj_t[1], N, NW, iters)

  count = _make_sc_agg(N, D, E_pad, mode="count")
  agg = _make_sc_agg(N, D, E_pad, mode="agg")

  dst_flat = dst.reshape(-1)
  (cnt,) = count(dst)
  (p,) = agg(x, src, dst_flat)
  h1 = _dense_layer(p, cnt, x, Wl0, bl0, Wr0, g0, be0, relu=True)
  (p,) = agg(h1, src, dst_flat)
  h2 = _dense_layer(p, cnt, h1, Wl1, bl1, Wr1, g1, be1, relu=True)
  (p,) = agg(h2, src, dst_flat)
  return _dense_final(p, cnt, h2, Wl2, bl2, Wr2, g2, be2, Wf, bf)


# speedup vs baseline: 1.9549x; 1.0927x over previous
import math

import jax
import jax.numpy as jnp
from jax import lax
from jax.experimental import pallas as pl
from jax.experimental.pallas import tpu as pltpu
from jax.experimental.pallas import tpu_sc as plsc

_BN_SCALE = 1.0 / math.sqrt(1.0 + 1e-5)
_K = 128
_NBUF = 2


def _sc_geometry(E):
  info = plsc.get_sparse_core_info()
  NC, NS = info.num_cores, info.num_subcores
  NW = NC * NS
  iters = -(-E // (_K * NW))
  return NC, NS, NW, iters, iters * _K * NW


def _make_sc_agg(N, D, E_pad, mode):
  NC, NS, NW, iters, e_chk = _sc_geometry(E_pad)
  assert e_chk == E_pad
  NA = N + 8
  rows_per_tile = (N // NS) & ~7
  rows_extra = N - rows_per_tile * NS
  assert rows_extra % 8 == 0 and rows_extra + 8 <= _K
  gather = mode == "agg"
  CH = iters + _NBUF

  mesh = plsc.VectorSubcoreMesh(core_axis_name="c", subcore_axis_name="s")
  out_type = [jax.ShapeDtypeStruct((NC, N, D), jnp.float32)]
  if gather:
    scratch = [
        pltpu.VMEM((CH, _K), jnp.int32),
        pltpu.VMEM((_K, D), jnp.float32),
        pltpu.VMEM((_K, D), jnp.float32),
        pltpu.VMEM((_K,), jnp.int32),
        pltpu.VMEM((_K,), jnp.int32),
        pltpu.VMEM((_K,), jnp.int32),
        pltpu.VMEM_SHARED((NA, D), jnp.float32),
        pltpu.SemaphoreType.DMA, pltpu.SemaphoreType.DMA,
        pltpu.SemaphoreType.DMA, pltpu.SemaphoreType.DMA,
        pltpu.SemaphoreType.DMA, pltpu.SemaphoreType.DMA,
        pltpu.SemaphoreType.DMA,
    ]
  else:
    scratch = [
        pltpu.VMEM((CH, _K), jnp.int32),
        pltpu.VMEM((_K, D), jnp.float32),
        pltpu.VMEM_SHARED((NA, D), jnp.float32),
        pltpu.SemaphoreType.DMA, pltpu.SemaphoreType.DMA,
    ]

  def body(*refs):
    if gather:
      (h_hbm, src_hbm, dst_hbm, out_hbm, src_v, r0, r1, db0, db1, db2, acc,
       gs0, gs1, ss0, ss1, ds0, ds1, ds2) = refs
      rows, dstb = [r0, r1], [db0, db1, db2]
      gsems, ssems, dsems = [gs0, gs1], [ss0, ss1], [ds0, ds1, ds2]
      rows0 = r0
    else:
      dst_hbm, out_hbm, dst_v, rows0, acc, cs0, cs1 = refs
      csems = [cs0, cs1]
    cid = lax.axis_index("c")
    sid = lax.axis_index("s")
    wid = sid * NC + cid

    if gather:
      pltpu.sync_copy(src_hbm.at[wid], src_v)
    else:
      pltpu.sync_copy(dst_hbm.at[wid], dst_v)

    def fill_rows(val):
      vec = jnp.full((16,), val, jnp.float32)

      def w(i, _):
        rows0[i // (D // 16), pl.ds((i % (D // 16)) * 16, 16)] = vec
        return 0
      lax.fori_loop(0, _K * (D // 16), w, 0)

    fill_rows(0.0)

    row0 = sid * rows_per_tile
    n_full = rows_per_tile // _K
    tail = rows_per_tile - n_full * _K
    for b in range(n_full):
      pltpu.sync_copy(rows0, acc.at[pl.ds(row0 + b * _K, _K)])
    if tail:
      pltpu.sync_copy(rows0.at[pl.ds(0, tail)],
                      acc.at[pl.ds(row0 + n_full * _K, tail)])

    @pl.when(sid == 0)
    def _():
      pltpu.sync_copy(rows0.at[pl.ds(0, rows_extra + 8)],
                      acc.at[pl.ds(rows_per_tile * NS, rows_extra + 8)])

    if not gather:
      fill_rows(1.0)
    plsc.subcore_barrier()

    if gather:
      def fire_g(c, rb):
        pltpu.async_copy(h_hbm.at[src_v.at[c]], rows[rb], gsems[rb])

      def fire_d(c, db3):
        pltpu.async_copy(dst_hbm.at[pl.ds((wid * CH + c) * _K, _K)],
                         dstb[db3], dsems[db3])

      def step(c, rb, db3, first=False, last=False):
        if not last:
          fire_d(c + 1, (db3 + 1) % 3)
        pltpu.make_async_copy(h_hbm.at[src_v.at[c]], rows[rb],
                              gsems[rb]).wait()
        pltpu.make_async_copy(dst_hbm.at[pl.ds((wid * CH + c) * _K, _K)],
                              dstb[db3], dsems[db3]).wait()
        pltpu.async_copy(rows[rb], acc.at[dstb[db3]], ssems[rb], add=True)
        if not first:
          pltpu.make_async_copy(rows[1 - rb], acc.at[dstb[(db3 + 2) % 3]],
                                ssems[1 - rb]).wait()
        if not last:
          fire_g(c + 1, 1 - rb)

      fire_d(0, 0)
      fire_g(0, 0)
      step(0, 0, 0, first=True)
      n_outer = (iters - 2) // 6

      def outer(g, _):
        for j in range(6):
          step(1 + g * 6 + j, (1 + j) % 2, (1 + j) % 3)
        return 0
      lax.fori_loop(0, n_outer, outer, 0)
      for c in range(1 + n_outer * 6, iters):
        step(c, c % 2, c % 3, last=(c == iters - 1))
      pltpu.make_async_copy(rows[(iters - 1) % 2],
                            acc.at[dstb[(iters - 1) % 3]],
                            ssems[(iters - 1) % 2]).wait()
    else:
      def cstep(c, b):
        pltpu.async_copy(rows0, acc.at[dst_v.at[c]], csems[b], add=True)
        pltpu.make_async_copy(rows0, acc.at[dst_v.at[c - 1]],
                              csems[1 - b]).wait()

      pltpu.async_copy(rows0, acc.at[dst_v.at[0]], csems[0], add=True)
      n_outer = (iters - 1) // 2

      def pair(g, _):
        cstep(1 + g * 2, 1)
        cstep(2 + g * 2, 0)
        return 0
      lax.fori_loop(0, n_outer, pair, 0)
      for c in range(1 + n_outer * 2, iters):
        cstep(c, c % 2)
      pltpu.make_async_copy(rows0, acc.at[dst_v.at[iters - 1]],
                            csems[(iters - 1) % 2]).wait()
    plsc.subcore_barrier()

    pltpu.sync_copy(acc.at[pl.ds(row0, rows_per_tile)],
                    out_hbm.at[cid, pl.ds(row0, rows_per_tile)])
    if rows_extra:
      @pl.when(sid == 0)
      def _():
        pltpu.sync_copy(acc.at[pl.ds(rows_per_tile * NS, rows_extra)],
                        out_hbm.at[cid, pl.ds(rows_per_tile * NS, rows_extra)])

  return pl.kernel(body, out_type=out_type, mesh=mesh, scratch_types=scratch)


def _dense_layer(p, cnt, h, Wl, bl, Wr, g, be, relu):
  N, D = h.shape
  TILE = 1000

  def body(p0_ref, p1_ref, c0_ref, c1_ref, h_ref, wl_ref, bl_ref, wr_ref,
           g_ref, be_ref, o_ref):
    cnt_t = c0_ref[:, :1] + c1_ref[:, :1]
    inv = 1.0 / jnp.maximum(cnt_t, 1.0)
    mean = (p0_ref[...] + p1_ref[...]) * inv
    acc = lax.dot_general(mean, wl_ref[...], (((1,), (1,)), ((), ())),
                          preferred_element_type=jnp.float32)
    acc = acc + lax.dot_general(h_ref[...], wr_ref[...],
                                (((1,), (1,)), ((), ())),
                                preferred_element_type=jnp.float32)
    acc = acc + bl_ref[...]
    if relu:
      acc = jnp.maximum(acc, 0.0)
    o_ref[...] = acc * (g_ref[...] * _BN_SCALE) + be_ref[...]

  grid = (N // TILE,)
  row_spec = pl.BlockSpec((TILE, D), lambda i: (i, 0))
  cnt_spec = pl.BlockSpec((TILE, D), lambda i: (i, 0))
  full = lambda shape: pl.BlockSpec(shape, lambda i: (0,) * len(shape))
  return pl.pallas_call(
      body,
      grid=grid,
      in_specs=[row_spec, row_spec, cnt_spec, cnt_spec, row_spec,
                full((D, D)), full((1, D)), full((D, D)),
                full((1, D)), full((1, D))],
      out_specs=row_spec,
      out_shape=jax.ShapeDtypeStruct((N, D), jnp.float32),
  )(p[0], p[1], cnt[0], cnt[1], h, Wl, bl.reshape(1, D), Wr,
    g.reshape(1, D), be.reshape(1, D))


def _dense_final(p, cnt, h, Wl, bl, Wr, g, be, Wf, bf):
  N, D = h.shape
  TILE = 1000

  def body(p0_ref, p1_ref, c0_ref, c1_ref, h_ref, wl_ref, bl_ref, wr_ref,
           g_ref, be_ref, wf_ref, bf_ref, o_ref):
    cnt_t = c0_ref[:, :1] + c1_ref[:, :1]
    inv = 1.0 / jnp.maximum(cnt_t, 1.0)
    mean = (p0_ref[...] + p1_ref[...]) * inv
    acc = lax.dot_general(mean, wl_ref[...], (((1,), (1,)), ((), ())),
                          preferred_element_type=jnp.float32)
    acc = acc + lax.dot_general(h_ref[...], wr_ref[...],
                                (((1,), (1,)), ((), ())),
                                preferred_element_type=jnp.float32)
    acc = acc + bl_ref[...]
    acc = acc * (g_ref[...] * _BN_SCALE) + be_ref[...]
    logit = jnp.sum(acc * wf_ref[...], axis=1, keepdims=True) + bf_ref[0, 0]
    o_ref[...] = 1.0 / (1.0 + jnp.exp(-logit))

  grid = (N // TILE,)
  row_spec = pl.BlockSpec((TILE, D), lambda i: (i, 0))
  cnt_spec = pl.BlockSpec((TILE, D), lambda i: (i, 0))
  full = lambda shape: pl.BlockSpec(shape, lambda i: (0,) * len(shape))
  return pl.pallas_call(
      body,
      grid=grid,
      in_specs=[row_spec, row_spec, cnt_spec, cnt_spec, row_spec,
                full((D, D)), full((1, D)), full((D, D)),
                full((1, D)), full((1, D)), full((1, D)),
                pl.BlockSpec(memory_space=pltpu.SMEM)],
      out_specs=pl.BlockSpec((TILE, 1), lambda i: (i, 0)),
      out_shape=jax.ShapeDtypeStruct((N, 1), jnp.float32),
  )(p[0], p[1], cnt[0], cnt[1], h, Wl, bl.reshape(1, D), Wr,
    g.reshape(1, D), be.reshape(1, D), Wf.reshape(1, D), bf.reshape(1, 1))


def _pad_edges(src, dst, N, NW, iters):
  pad = NW * iters * _K - src.shape[0]
  src_p = jnp.concatenate([src, jnp.zeros((pad,), jnp.int32)])
  dst_p = jnp.concatenate(
      [dst, N + (jnp.arange(pad, dtype=jnp.int32) % 8)])
  src_p = src_p.reshape(iters, NW, _K).transpose(1, 0, 2)
  dst_p = dst_p.reshape(iters, NW, _K).transpose(1, 0, 2)
  src_p = jnp.concatenate(
      [src_p, jnp.zeros((NW, _NBUF, _K), jnp.int32)], axis=1)
  dst_p = jnp.concatenate(
      [dst_p, jnp.full((NW, _NBUF, _K), N, jnp.int32)], axis=1)
  return src_p, dst_p


@jax.jit
def kernel(x, adj_t, Wl0, bl0, Wr0, Wl1, bl1, Wr1, Wl2, bl2, Wr2,
           g0, be0, g1, be1, g2, be2, Wf, bf):
  N, D = x.shape
  E = adj_t.shape[1]
  _, _, NW, iters, E_pad = _sc_geometry(E)
  src, dst = _pad_edges(adj_t[0], adj_t[1], N, NW, iters)

  count = _make_sc_agg(N, D, E_pad, mode="count")
  agg = _make_sc_agg(N, D, E_pad, mode="agg")

  dst_flat = dst.reshape(-1)
  (cnt,) = count(dst)
  (p,) = agg(x, src, dst_flat)
  h1 = _dense_layer(p, cnt, x, Wl0, bl0, Wr0, g0, be0, relu=True)
  (p,) = agg(h1, src, dst_flat)
  h2 = _dense_layer(p, cnt, h1, Wl1, bl1, Wr1, g1, be1, relu=True)
  (p,) = agg(h2, src, dst_flat)
  return _dense_final(p, cnt, h2, Wl2, bl2, Wr2, g2, be2, Wf, bf)


# speedup vs baseline: 1.9578x; 1.0015x over previous
import math

import jax
import jax.numpy as jnp
from jax import lax
from jax.experimental import pallas as pl
from jax.experimental.pallas import tpu as pltpu
from jax.experimental.pallas import tpu_sc as plsc

_BN_SCALE = 1.0 / math.sqrt(1.0 + 1e-5)
_K = 128
_NBUF = 2


def _sc_geometry(E):
  info = plsc.get_sparse_core_info()
  NC, NS = info.num_cores, info.num_subcores
  NW = NC * NS
  iters = -(-E // (_K * NW))
  return NC, NS, NW, iters, iters * _K * NW


def _make_sc_agg(N, D, E_pad, mode):
  NC, NS, NW, iters, e_chk = _sc_geometry(E_pad)
  assert e_chk == E_pad
  NA = N + 8
  rows_per_tile = (N // NS) & ~7
  rows_extra = N - rows_per_tile * NS
  assert rows_extra % 8 == 0 and rows_extra + 8 <= _K
  gather = mode == "agg"
  CH = iters + _NBUF

  mesh = plsc.VectorSubcoreMesh(core_axis_name="c", subcore_axis_name="s")
  out_type = [jax.ShapeDtypeStruct((NC, N, D), jnp.float32)]
  if gather:
    scratch = [
        pltpu.VMEM((_K, D), jnp.float32),
        pltpu.VMEM((_K, D), jnp.float32),
        pltpu.VMEM((_K, D), jnp.float32),
        pltpu.VMEM((_K,), jnp.int32),
        pltpu.VMEM((_K,), jnp.int32),
        pltpu.VMEM((_K,), jnp.int32),
        pltpu.VMEM((_K,), jnp.int32),
        pltpu.VMEM((_K,), jnp.int32),
        pltpu.VMEM((_K,), jnp.int32),
        pltpu.VMEM_SHARED((NA, D), jnp.float32),
    ] + [pltpu.SemaphoreType.DMA] * 9
  else:
    scratch = [
        pltpu.VMEM((CH, _K), jnp.int32),
        pltpu.VMEM((_K, D), jnp.float32),
        pltpu.VMEM_SHARED((NA, D), jnp.float32),
        pltpu.SemaphoreType.DMA, pltpu.SemaphoreType.DMA,
    ]

  def body(*refs):
    if gather:
      (h_hbm, src_hbm, dst_hbm, out_hbm, r0, r1, r2, sb0, sb1, sb2,
       db0, db1, db2, acc, gs0, gs1, gs2, ss0, ss1, ss2,
       is0, is1, is2) = refs
      rows, srcb, dstb = [r0, r1, r2], [sb0, sb1, sb2], [db0, db1, db2]
      gsems, ssems, isems = [gs0, gs1, gs2], [ss0, ss1, ss2], [is0, is1, is2]
      rows0 = r0
    else:
      dst_hbm, out_hbm, dst_v, rows0, acc, cs0, cs1 = refs
      csems = [cs0, cs1]
    cid = lax.axis_index("c")
    sid = lax.axis_index("s")
    wid = sid * NC + cid

    if not gather:
      pltpu.sync_copy(dst_hbm.at[wid], dst_v)

    def fill_rows(val):
      vec = jnp.full((16,), val, jnp.float32)

      def w(i, _):
        rows0[i // (D // 16), pl.ds((i % (D // 16)) * 16, 16)] = vec
        return 0
      lax.fori_loop(0, _K * (D // 16), w, 0)

    fill_rows(0.0)

    row0 = sid * rows_per_tile
    n_full = rows_per_tile // _K
    tail = rows_per_tile - n_full * _K
    for b in range(n_full):
      pltpu.sync_copy(rows0, acc.at[pl.ds(row0 + b * _K, _K)])
    if tail:
      pltpu.sync_copy(rows0.at[pl.ds(0, tail)],
                      acc.at[pl.ds(row0 + n_full * _K, tail)])

    @pl.when(sid == 0)
    def _():
      pltpu.sync_copy(rows0.at[pl.ds(0, rows_extra + 8)],
                      acc.at[pl.ds(rows_per_tile * NS, rows_extra + 8)])

    if not gather:
      fill_rows(1.0)
    plsc.subcore_barrier()

    if gather:
      def fire_idx(c, m3):
        pltpu.async_copy(dst_hbm.at[pl.ds((wid * CH + c) * _K, _K)],
                         dstb[m3], isems[m3])
        pltpu.async_copy(src_hbm.at[pl.ds((wid * CH + c) * _K, _K)],
                         srcb[m3], isems[m3])

      def wait_idx(c, m3):
        pltpu.make_async_copy(dst_hbm.at[pl.ds((wid * CH + c) * _K, _K)],
                              dstb[m3], isems[m3]).wait()
        pltpu.make_async_copy(src_hbm.at[pl.ds((wid * CH + c) * _K, _K)],
                              srcb[m3], isems[m3]).wait()

      def fire_g(c, m3):
        pltpu.async_copy(h_hbm.at[srcb[m3]], rows[m3], gsems[m3])

      def wait_g(m3):
        pltpu.make_async_copy(h_hbm.at[srcb[m3]], rows[m3],
                              gsems[m3]).wait()

      def fire_s(m3):
        pltpu.async_copy(rows[m3], acc.at[dstb[m3]], ssems[m3], add=True)

      def wait_s(m3):
        pltpu.make_async_copy(rows[m3], acc.at[dstb[m3]], ssems[m3]).wait()

      def step(c, m3, first=False, last=False):
        if not last:
          fire_idx(c + 1, (m3 + 1) % 3)
        wait_g(m3)
        fire_s(m3)
        if not first:
          wait_s((m3 + 1) % 3)
        if not last:
          wait_idx(c + 1, (m3 + 1) % 3)
          fire_g(c + 1, (m3 + 1) % 3)

      fire_idx(0, 0)
      wait_idx(0, 0)
      fire_g(0, 0)
      step(0, 0, first=True)
      step(1, 1, first=True)
      n_outer = (iters - 4) // 3

      def outer(g, _):
        for j in range(3):
          step(2 + g * 3 + j, (2 + j) % 3)
        return 0
      lax.fori_loop(0, n_outer, outer, 0)
      for c in range(2 + n_outer * 3, iters):
        step(c, c % 3, last=(c == iters - 1))
      wait_s((iters - 2) % 3)
      wait_s((iters - 1) % 3)
    else:
      def cstep(c, b):
        pltpu.async_copy(rows0, acc.at[dst_v.at[c]], csems[b], add=True)
        pltpu.make_async_copy(rows0, acc.at[dst_v.at[c - 1]],
                              csems[1 - b]).wait()

      pltpu.async_copy(rows0, acc.at[dst_v.at[0]], csems[0], add=True)
      n_outer = (iters - 1) // 2

      def pair(g, _):
        cstep(1 + g * 2, 1)
        cstep(2 + g * 2, 0)
        return 0
      lax.fori_loop(0, n_outer, pair, 0)
      for c in range(1 + n_outer * 2, iters):
        cstep(c, c % 2)
      pltpu.make_async_copy(rows0, acc.at[dst_v.at[iters - 1]],
                            csems[(iters - 1) % 2]).wait()
    plsc.subcore_barrier()

    pltpu.sync_copy(acc.at[pl.ds(row0, rows_per_tile)],
                    out_hbm.at[cid, pl.ds(row0, rows_per_tile)])
    if rows_extra:
      @pl.when(sid == 0)
      def _():
        pltpu.sync_copy(acc.at[pl.ds(rows_per_tile * NS, rows_extra)],
                        out_hbm.at[cid, pl.ds(rows_per_tile * NS, rows_extra)])

  return pl.kernel(body, out_type=out_type, mesh=mesh, scratch_types=scratch)


def _dense_layer(p, cnt, h, Wl, bl, Wr, g, be, relu):
  N, D = h.shape
  TILE = 1000

  def body(p0_ref, p1_ref, c0_ref, c1_ref, h_ref, wl_ref, bl_ref, wr_ref,
           g_ref, be_ref, o_ref):
    cnt_t = c0_ref[:, :1] + c1_ref[:, :1]
    inv = 1.0 / jnp.maximum(cnt_t, 1.0)
    mean = (p0_ref[...] + p1_ref[...]) * inv
    acc = lax.dot_general(mean, wl_ref[...], (((1,), (1,)), ((), ())),
                          preferred_element_type=jnp.float32)
    acc = acc + lax.dot_general(h_ref[...], wr_ref[...],
                                (((1,), (1,)), ((), ())),
                                preferred_element_type=jnp.float32)
    acc = acc + bl_ref[...]
    if relu:
      acc = jnp.maximum(acc, 0.0)
    o_ref[...] = acc * (g_ref[...] * _BN_SCALE) + be_ref[...]

  grid = (N // TILE,)
  row_spec = pl.BlockSpec((TILE, D), lambda i: (i, 0))
  cnt_spec = pl.BlockSpec((TILE, D), lambda i: (i, 0))
  full = lambda shape: pl.BlockSpec(shape, lambda i: (0,) * len(shape))
  return pl.pallas_call(
      body,
      grid=grid,
      in_specs=[row_spec, row_spec, cnt_spec, cnt_spec, row_spec,
                full((D, D)), full((1, D)), full((D, D)),
                full((1, D)), full((1, D))],
      out_specs=row_spec,
      out_shape=jax.ShapeDtypeStruct((N, D), jnp.float32),
  )(p[0], p[1], cnt[0], cnt[1], h, Wl, bl.reshape(1, D), Wr,
    g.reshape(1, D), be.reshape(1, D))


def _dense_final(p, cnt, h, Wl, bl, Wr, g, be, Wf, bf):
  N, D = h.shape
  TILE = 1000

  def body(p0_ref, p1_ref, c0_ref, c1_ref, h_ref, wl_ref, bl_ref, wr_ref,
           g_ref, be_ref, wf_ref, bf_ref, o_ref):
    cnt_t = c0_ref[:, :1] + c1_ref[:, :1]
    inv = 1.0 / jnp.maximum(cnt_t, 1.0)
    mean = (p0_ref[...] + p1_ref[...]) * inv
    acc = lax.dot_general(mean, wl_ref[...], (((1,), (1,)), ((), ())),
                          preferred_element_type=jnp.float32)
    acc = acc + lax.dot_general(h_ref[...], wr_ref[...],
                                (((1,), (1,)), ((), ())),
                                preferred_element_type=jnp.float32)
    acc = acc + bl_ref[...]
    acc = acc * (g_ref[...] * _BN_SCALE) + be_ref[...]
    logit = jnp.sum(acc * wf_ref[...], axis=1, keepdims=True) + bf_ref[0, 0]
    o_ref[...] = 1.0 / (1.0 + jnp.exp(-logit))

  grid = (N // TILE,)
  row_spec = pl.BlockSpec((TILE, D), lambda i: (i, 0))
  cnt_spec = pl.BlockSpec((TILE, D), lambda i: (i, 0))
  full = lambda shape: pl.BlockSpec(shape, lambda i: (0,) * len(shape))
  return pl.pallas_call(
      body,
      grid=grid,
      in_specs=[row_spec, row_spec, cnt_spec, cnt_spec, row_spec,
                full((D, D)), full((1, D)), full((D, D)),
                full((1, D)), full((1, D)), full((1, D)),
                pl.BlockSpec(memory_space=pltpu.SMEM)],
      out_specs=pl.BlockSpec((TILE, 1), lambda i: (i, 0)),
      out_shape=jax.ShapeDtypeStruct((N, 1), jnp.float32),
  )(p[0], p[1], cnt[0], cnt[1], h, Wl, bl.reshape(1, D), Wr,
    g.reshape(1, D), be.reshape(1, D), Wf.reshape(1, D), bf.reshape(1, 1))


def _pad_edges(src, dst, N, NW, iters):
  pad = NW * iters * _K - src.shape[0]
  src_p = jnp.concatenate([src, jnp.zeros((pad,), jnp.int32)])
  dst_p = jnp.concatenate(
      [dst, N + (jnp.arange(pad, dtype=jnp.int32) % 8)])
  src_p = src_p.reshape(iters, NW, _K).transpose(1, 0, 2)
  dst_p = dst_p.reshape(iters, NW, _K).transpose(1, 0, 2)
  src_p = jnp.concatenate(
      [src_p, jnp.zeros((NW, _NBUF, _K), jnp.int32)], axis=1)
  dst_p = jnp.concatenate(
      [dst_p, jnp.full((NW, _NBUF, _K), N, jnp.int32)], axis=1)
  return src_p, dst_p


@jax.jit
def kernel(x, adj_t, Wl0, bl0, Wr0, Wl1, bl1, Wr1, Wl2, bl2, Wr2,
           g0, be0, g1, be1, g2, be2, Wf, bf):
  N, D = x.shape
  E = adj_t.shape[1]
  _, _, NW, iters, E_pad = _sc_geometry(E)
  src, dst = _pad_edges(adj_t[0], adj_t[1], N, NW, iters)

  count = _make_sc_agg(N, D, E_pad, mode="count")
  agg = _make_sc_agg(N, D, E_pad, mode="agg")

  src_flat = src.reshape(-1)
  dst_flat = dst.reshape(-1)
  (cnt,) = count(dst)
  (p,) = agg(x, src_flat, dst_flat)
  h1 = _dense_layer(p, cnt, x, Wl0, bl0, Wr0, g0, be0, relu=True)
  (p,) = agg(h1, src_flat, dst_flat)
  h2 = _dense_layer(p, cnt, h1, Wl1, bl1, Wr1, g1, be1, relu=True)
  (p,) = agg(h2, src_flat, dst_flat)
  return _dense_final(p, cnt, h2, Wl2, bl2, Wr2, g2, be2, Wf, bf)


# speedup vs baseline: 1.9755x; 1.0091x over previous
import math

import jax
import jax.numpy as jnp
from jax import lax
from jax.experimental import pallas as pl
from jax.experimental.pallas import tpu as pltpu
from jax.experimental.pallas import tpu_sc as plsc

_BN_SCALE = 1.0 / math.sqrt(1.0 + 1e-5)
_K = 128
_NBUF = 2


def _sc_geometry(E):
  info = plsc.get_sparse_core_info()
  NC, NS = info.num_cores, info.num_subcores
  NW = NC * NS
  iters = -(-E // (_K * NW))
  return NC, NS, NW, iters, iters * _K * NW


def _make_sc_agg(N, D, E_pad, mode):
  NC, NS, NW, iters, e_chk = _sc_geometry(E_pad)
  assert e_chk == E_pad
  NA = N + 8
  rows_per_tile = (N // NS) & ~7
  rows_extra = N - rows_per_tile * NS
  assert rows_extra % 8 == 0 and rows_extra + 8 <= _K
  gather = mode == "agg"
  CH = iters + _NBUF

  mesh = plsc.VectorSubcoreMesh(core_axis_name="c", subcore_axis_name="s")
  out_type = [jax.ShapeDtypeStruct((NC, N, D), jnp.float32)]
  if gather:
    scratch = [
        pltpu.VMEM((_K, D), jnp.float32),
        pltpu.VMEM((_K, D), jnp.float32),
        pltpu.VMEM((_K, D), jnp.float32),
        pltpu.VMEM((2, _K), jnp.int32),
        pltpu.VMEM((2, _K), jnp.int32),
        pltpu.VMEM((2, _K), jnp.int32),
        pltpu.VMEM_SHARED((NA, D), jnp.float32),
    ] + [pltpu.SemaphoreType.DMA] * 9
  else:
    scratch = [
        pltpu.VMEM((CH, _K), jnp.int32),
        pltpu.VMEM((_K, D), jnp.float32),
        pltpu.VMEM_SHARED((NA, D), jnp.float32),
        pltpu.SemaphoreType.DMA, pltpu.SemaphoreType.DMA,
    ]

  def body(*refs):
    if gather:
      (h_hbm, idx_hbm, out_hbm, r0, r1, r2, ib0, ib1, ib2, acc,
       gs0, gs1, gs2, ss0, ss1, ss2, is0, is1, is2) = refs
      rows, idxb = [r0, r1, r2], [ib0, ib1, ib2]
      gsems, ssems, isems = [gs0, gs1, gs2], [ss0, ss1, ss2], [is0, is1, is2]
      rows0 = r0
    else:
      dst_hbm, out_hbm, dst_v, rows0, acc, cs0, cs1 = refs
      csems = [cs0, cs1]
    cid = lax.axis_index("c")
    sid = lax.axis_index("s")
    wid = sid * NC + cid

    if not gather:
      pltpu.sync_copy(dst_hbm.at[wid], dst_v)

    def fill_rows(val):
      vec = jnp.full((16,), val, jnp.float32)

      def w(i, _):
        rows0[i // (D // 16), pl.ds((i % (D // 16)) * 16, 16)] = vec
        return 0
      lax.fori_loop(0, _K * (D // 16), w, 0)

    fill_rows(0.0)

    row0 = sid * rows_per_tile
    n_full = rows_per_tile // _K
    tail = rows_per_tile - n_full * _K
    for b in range(n_full):
      pltpu.sync_copy(rows0, acc.at[pl.ds(row0 + b * _K, _K)])
    if tail:
      pltpu.sync_copy(rows0.at[pl.ds(0, tail)],
                      acc.at[pl.ds(row0 + n_full * _K, tail)])

    @pl.when(sid == 0)
    def _():
      pltpu.sync_copy(rows0.at[pl.ds(0, rows_extra + 8)],
                      acc.at[pl.ds(rows_per_tile * NS, rows_extra + 8)])

    if not gather:
      fill_rows(1.0)
    plsc.subcore_barrier()

    if gather:
      def fire_idx(c, m3):
        pltpu.async_copy(idx_hbm.at[wid * CH + c], idxb[m3], isems[m3])

      def wait_idx(c, m3):
        pltpu.make_async_copy(idx_hbm.at[wid * CH + c], idxb[m3],
                              isems[m3]).wait()

      def fire_g(c, m3):
        pltpu.async_copy(h_hbm.at[idxb[m3].at[0]], rows[m3], gsems[m3])

      def wait_g(m3):
        pltpu.make_async_copy(h_hbm.at[idxb[m3].at[0]], rows[m3],
                              gsems[m3]).wait()

      def fire_s(m3):
        pltpu.async_copy(rows[m3], acc.at[idxb[m3].at[1]], ssems[m3],
                         add=True)

      def wait_s(m3):
        pltpu.make_async_copy(rows[m3], acc.at[idxb[m3].at[1]],
                              ssems[m3]).wait()

      def step(c, m3, first=False, last=False):
        if not last:
          fire_idx(c + 1, (m3 + 1) % 3)
        wait_g(m3)
        fire_s(m3)
        if not first:
          wait_s((m3 + 1) % 3)
        if not last:
          wait_idx(c + 1, (m3 + 1) % 3)
          fire_g(c + 1, (m3 + 1) % 3)

      fire_idx(0, 0)
      wait_idx(0, 0)
      fire_g(0, 0)
      step(0, 0, first=True)
      step(1, 1, first=True)
      n_outer = (iters - 4) // 3

      def outer(g, _):
        for j in range(3):
          step(2 + g * 3 + j, (2 + j) % 3)
        return 0
      lax.fori_loop(0, n_outer, outer, 0)
      for c in range(2 + n_outer * 3, iters):
        step(c, c % 3, last=(c == iters - 1))
      wait_s((iters - 2) % 3)
      wait_s((iters - 1) % 3)
    else:
      def cstep(c, b):
        pltpu.async_copy(rows0, acc.at[dst_v.at[c]], csems[b], add=True)
        pltpu.make_async_copy(rows0, acc.at[dst_v.at[c - 1]],
                              csems[1 - b]).wait()

      pltpu.async_copy(rows0, acc.at[dst_v.at[0]], csems[0], add=True)
      n_outer = (iters - 1) // 2

      def pair(g, _):
        cstep(1 + g * 2, 1)
        cstep(2 + g * 2, 0)
        return 0
      lax.fori_loop(0, n_outer, pair, 0)
      for c in range(1 + n_outer * 2, iters):
        cstep(c, c % 2)
      pltpu.make_async_copy(rows0, acc.at[dst_v.at[iters - 1]],
                            csems[(iters - 1) % 2]).wait()
    plsc.subcore_barrier()

    pltpu.sync_copy(acc.at[pl.ds(row0, rows_per_tile)],
                    out_hbm.at[cid, pl.ds(row0, rows_per_tile)])
    if rows_extra:
      @pl.when(sid == 0)
      def _():
        pltpu.sync_copy(acc.at[pl.ds(rows_per_tile * NS, rows_extra)],
                        out_hbm.at[cid, pl.ds(rows_per_tile * NS, rows_extra)])

  return pl.kernel(body, out_type=out_type, mesh=mesh, scratch_types=scratch)


def _dense_layer(p, cnt, h, Wl, bl, Wr, g, be, relu):
  N, D = h.shape
  TILE = 2000

  def body(p0_ref, p1_ref, c0_ref, c1_ref, h_ref, wl_ref, bl_ref, wr_ref,
           g_ref, be_ref, o_ref):
    cnt_t = c0_ref[:, :1] + c1_ref[:, :1]
    inv = 1.0 / jnp.maximum(cnt_t, 1.0)
    mean = (p0_ref[...] + p1_ref[...]) * inv
    acc = lax.dot_general(mean, wl_ref[...], (((1,), (1,)), ((), ())),
                          preferred_element_type=jnp.float32)
    acc = acc + lax.dot_general(h_ref[...], wr_ref[...],
                                (((1,), (1,)), ((), ())),
                                preferred_element_type=jnp.float32)
    acc = acc + bl_ref[...]
    if relu:
      acc = jnp.maximum(acc, 0.0)
    o_ref[...] = acc * (g_ref[...] * _BN_SCALE) + be_ref[...]

  grid = (N // TILE,)
  row_spec = pl.BlockSpec((TILE, D), lambda i: (i, 0))
  cnt_spec = pl.BlockSpec((TILE, D), lambda i: (i, 0))
  full = lambda shape: pl.BlockSpec(shape, lambda i: (0,) * len(shape))
  return pl.pallas_call(
      body,
      grid=grid,
      in_specs=[row_spec, row_spec, cnt_spec, cnt_spec, row_spec,
                full((D, D)), full((1, D)), full((D, D)),
                full((1, D)), full((1, D))],
      out_specs=row_spec,
      out_shape=jax.ShapeDtypeStruct((N, D), jnp.float32),
  )(p[0], p[1], cnt[0], cnt[1], h, Wl, bl.reshape(1, D), Wr,
    g.reshape(1, D), be.reshape(1, D))


def _dense_final(p, cnt, h, Wl, bl, Wr, g, be, Wf, bf):
  N, D = h.shape
  TILE = 2000

  def body(p0_ref, p1_ref, c0_ref, c1_ref, h_ref, wl_ref, bl_ref, wr_ref,
           g_ref, be_ref, wf_ref, bf_ref, o_ref):
    cnt_t = c0_ref[:, :1] + c1_ref[:, :1]
    inv = 1.0 / jnp.maximum(cnt_t, 1.0)
    mean = (p0_ref[...] + p1_ref[...]) * inv
    acc = lax.dot_general(mean, wl_ref[...], (((1,), (1,)), ((), ())),
                          preferred_element_type=jnp.float32)
    acc = acc + lax.dot_general(h_ref[...], wr_ref[...],
                                (((1,), (1,)), ((), ())),
                                preferred_element_type=jnp.float32)
    acc = acc + bl_ref[...]
    acc = acc * (g_ref[...] * _BN_SCALE) + be_ref[...]
    logit = jnp.sum(acc * wf_ref[...], axis=1, keepdims=True) + bf_ref[0, 0]
    o_ref[...] = 1.0 / (1.0 + jnp.exp(-logit))

  grid = (N // TILE,)
  row_spec = pl.BlockSpec((TILE, D), lambda i: (i, 0))
  cnt_spec = pl.BlockSpec((TILE, D), lambda i: (i, 0))
  full = lambda shape: pl.BlockSpec(shape, lambda i: (0,) * len(shape))
  return pl.pallas_call(
      body,
      grid=grid,
      in_specs=[row_spec, row_spec, cnt_spec, cnt_spec, row_spec,
                full((D, D)), full((1, D)), full((D, D)),
                full((1, D)), full((1, D)), full((1, D)),
                pl.BlockSpec(memory_space=pltpu.SMEM)],
      out_specs=pl.BlockSpec((TILE, 1), lambda i: (i, 0)),
      out_shape=jax.ShapeDtypeStruct((N, 1), jnp.float32),
  )(p[0], p[1], cnt[0], cnt[1], h, Wl, bl.reshape(1, D), Wr,
    g.reshape(1, D), be.reshape(1, D), Wf.reshape(1, D), bf.reshape(1, 1))


def _pad_edges(src, dst, N, NW, iters):
  pad = NW * iters * _K - src.shape[0]
  src_p = jnp.concatenate([src, jnp.zeros((pad,), jnp.int32)])
  dst_p = jnp.concatenate(
      [dst, N + (jnp.arange(pad, dtype=jnp.int32) % 8)])
  src_p = src_p.reshape(iters, NW, _K).transpose(1, 0, 2)
  dst_p = dst_p.reshape(iters, NW, _K).transpose(1, 0, 2)
  src_p = jnp.concatenate(
      [src_p, jnp.zeros((NW, _NBUF, _K), jnp.int32)], axis=1)
  dst_p = jnp.concatenate(
      [dst_p, jnp.full((NW, _NBUF, _K), N, jnp.int32)], axis=1)
  packed = jnp.stack([src_p, dst_p], axis=2).reshape(-1, 2, _K)
  return src_p, dst_p, packed


@jax.jit
def kernel(x, adj_t, Wl0, bl0, Wr0, Wl1, bl1, Wr1, Wl2, bl2, Wr2,
           g0, be0, g1, be1, g2, be2, Wf, bf):
  N, D = x.shape
  E = adj_t.shape[1]
  _, _, NW, iters, E_pad = _sc_geometry(E)
  src, dst, packed = _pad_edges(adj_t[0], adj_t[1], N, NW, iters)

  count = _make_sc_agg(N, D, E_pad, mode="count")
  agg = _make_sc_agg(N, D, E_pad, mode="agg")

  (cnt,) = count(dst)
  (p,) = agg(x, packed)
  h1 = _dense_layer(p, cnt, x, Wl0, bl0, Wr0, g0, be0, relu=True)
  (p,) = agg(h1, packed)
  h2 = _dense_layer(p, cnt, h1, Wl1, bl1, Wr1, g1, be1, relu=True)
  (p,) = agg(h2, packed)
  return _dense_final(p, cnt, h2, Wl2, bl2, Wr2, g2, be2, Wf, bf)
